# baseline (speedup 1.0000x reference)
import jax
import jax.numpy as jnp
from jax import lax
from jax.experimental import pallas as pl

N_EXPERTS = 64
TOP_K = 2
BLOCK_N = 2048


def _router_kernel(x_ref, w_ref, b_ref, out_ref):
    logits = jnp.dot(
        x_ref[...], w_ref[...], preferred_element_type=jnp.float32
    ) + b_ref[...]

    e = lax.broadcasted_iota(jnp.int32, logits.shape, 1)

    m1 = jnp.max(logits, axis=1, keepdims=True)
    i1 = jnp.min(jnp.where(logits == m1, e, N_EXPERTS), axis=1, keepdims=True)

    masked = jnp.where(e == i1, -jnp.inf, logits)
    m2 = jnp.max(masked, axis=1, keepdims=True)
    i2 = jnp.min(jnp.where(masked == m2, e, N_EXPERTS), axis=1, keepdims=True)

    e2 = jnp.exp(m2 - m1)
    denom = 1.0 + e2
    p1 = 1.0 / denom
    p2 = e2 / denom

    gates = jnp.where(e == i1, p1, 0.0) + jnp.where(e == i2, p2, 0.0)

    idx_f = lax.bitcast_convert_type(
        jnp.where((e & 1) == 0, i1, i2), jnp.float32
    )
    out_ref[...] = jnp.concatenate([gates, idx_f], axis=1)


@jax.jit
def kernel(x, W, b):
    n, d = x.shape
    grid = (n // BLOCK_N,)
    out = pl.pallas_call(
        _router_kernel,
        grid=grid,
        in_specs=[
            pl.BlockSpec((BLOCK_N, d), lambda i: (i, 0)),
            pl.BlockSpec((d, N_EXPERTS), lambda i: (0, 0)),
            pl.BlockSpec((1, N_EXPERTS), lambda i: (0, 0)),
        ],
        out_specs=pl.BlockSpec((BLOCK_N, 2 * N_EXPERTS), lambda i: (i, 0)),
        out_shape=jax.ShapeDtypeStruct((n, 2 * N_EXPERTS), jnp.float32),
    )(x, W, b.reshape(1, N_EXPERTS))
    gates = lax.slice(out, (0, 0), (n, N_EXPERTS))
    idx = lax.bitcast_convert_type(
        lax.slice(out, (0, N_EXPERTS), (n, N_EXPERTS + TOP_K)), jnp.int32
    )
    return (gates, idx)


# speedup vs baseline: 1.2619x; 1.2619x over previous
import jax
import jax.numpy as jnp
from jax import lax
from jax.experimental import pallas as pl
from jax.experimental.pallas import tpu as pltpu

N_EXPERTS = 64
TOP_K = 2
BLOCK_N = 2048


def _router_kernel(x_ref, w_ref, b_ref, gates_ref, idx_ref):
    logits = jnp.dot(
        x_ref[...], w_ref[...], preferred_element_type=jnp.float32
    ) + b_ref[...]

    e = lax.broadcasted_iota(jnp.int32, logits.shape, 1)

    m1 = jnp.max(logits, axis=1, keepdims=True)
    i1 = jnp.min(jnp.where(logits == m1, e, N_EXPERTS), axis=1, keepdims=True)

    masked = jnp.where(e == i1, -jnp.inf, logits)
    m2 = jnp.max(masked, axis=1, keepdims=True)
    i2 = jnp.min(jnp.where(masked == m2, e, N_EXPERTS), axis=1, keepdims=True)

    e2 = jnp.exp(m2 - m1)
    denom = 1.0 + e2
    p1 = 1.0 / denom
    p2 = e2 / denom

    gates_ref[...] = jnp.where(e == i1, p1, 0.0) + jnp.where(e == i2, p2, 0.0)
    idx_ref[...] = jnp.concatenate([i1, i2], axis=1)


@jax.jit
def kernel(x, W, b):
    n, d = x.shape
    grid = (n // BLOCK_N,)
    gates, idx = pl.pallas_call(
        _router_kernel,
        grid=grid,
        in_specs=[
            pl.BlockSpec((BLOCK_N, d), lambda i: (i, 0)),
            pl.BlockSpec((d, N_EXPERTS), lambda i: (0, 0)),
            pl.BlockSpec((1, N_EXPERTS), lambda i: (0, 0)),
        ],
        out_specs=[
            pl.BlockSpec((BLOCK_N, N_EXPERTS), lambda i: (i, 0)),
            pl.BlockSpec((BLOCK_N, TOP_K), lambda i: (i, 0)),
        ],
        out_shape=[
            jax.ShapeDtypeStruct((n, N_EXPERTS), jnp.float32),
            jax.ShapeDtypeStruct((n, TOP_K), jnp.int32),
        ],
        compiler_params=pltpu.CompilerParams(
            dimension_semantics=("parallel",),
        ),
    )(x, W, b.reshape(1, N_EXPERTS))
    return (gates, idx)


# speedup vs baseline: 1.3100x; 1.0381x over previous
import jax
import jax.numpy as jnp
from jax import lax
from jax.experimental import pallas as pl
from jax.experimental.pallas import tpu as pltpu

N_EXPERTS = 64
TOP_K = 2
BLOCK_N = 2048


def _router_kernel(x_ref, w_ref, b_ref, gates_ref, idx_ref):
    logits = lax.dot_general(
        x_ref[...],
        w_ref[...],
        dimension_numbers=(((1,), (1,)), ((), ())),
        preferred_element_type=jnp.float32,
    ) + b_ref[...]

    e = lax.broadcasted_iota(jnp.int32, logits.shape, 1)

    m1 = jnp.max(logits, axis=1, keepdims=True)
    i1 = jnp.min(jnp.where(logits == m1, e, N_EXPERTS), axis=1, keepdims=True)

    masked = jnp.where(e == i1, -jnp.inf, logits)
    m2 = jnp.max(masked, axis=1, keepdims=True)
    i2 = jnp.min(jnp.where(masked == m2, e, N_EXPERTS), axis=1, keepdims=True)

    e2 = jnp.exp(m2 - m1)
    denom = 1.0 + e2
    p1 = 1.0 / denom
    p2 = e2 / denom

    gates_ref[...] = jnp.where(e == i1, p1, 0.0) + jnp.where(e == i2, p2, 0.0)
    idx_ref[...] = jnp.concatenate([i1, i2], axis=1)


@jax.jit
def kernel(x, W, b):
    n, d = x.shape
    grid = (n // BLOCK_N,)
    gates, idx = pl.pallas_call(
        _router_kernel,
        grid=grid,
        in_specs=[
            pl.BlockSpec((BLOCK_N, d), lambda i: (i, 0)),
            pl.BlockSpec((N_EXPERTS, d), lambda i: (0, 0)),
            pl.BlockSpec((1, N_EXPERTS), lambda i: (0, 0)),
        ],
        out_specs=[
            pl.BlockSpec((BLOCK_N, N_EXPERTS), lambda i: (i, 0)),
            pl.BlockSpec((BLOCK_N, TOP_K), lambda i: (i, 0)),
        ],
        out_shape=[
            jax.ShapeDtypeStruct((n, N_EXPERTS), jnp.float32),
            jax.ShapeDtypeStruct((n, TOP_K), jnp.int32),
        ],
        compiler_params=pltpu.CompilerParams(
            dimension_semantics=("parallel",),
        ),
    )(x, W.T, b.reshape(1, N_EXPERTS))
    return (gates, idx)
